# baseline (speedup 1.0000x reference)
import functools

import jax
import jax.numpy as jnp
from jax.experimental import pallas as pl
from jax.experimental.pallas import tpu as pltpu


def _router_body(x_ref, gw_ref, sgw_ref, gv_ref, waug_ref):
    x = x_ref[...]
    gw = gw_ref[...]
    gv = jax.lax.dot_general(x, gw, (((1,), (1,)), ((), ())),
                             preferred_element_type=jnp.float32)
    E = gv.shape[1]
    m = jnp.max(gv, axis=1, keepdims=True)
    p = jnp.exp(gv - m)
    s = p / jnp.sum(p, axis=1, keepdims=True)
    lane = jax.lax.broadcasted_iota(jnp.int32, gv.shape, 1)
    mx1 = jnp.max(gv, axis=1, keepdims=True)
    eq1 = gv == mx1
    i1 = jnp.min(jnp.where(eq1, lane, E), axis=1, keepdims=True)
    sel1 = lane == i1
    neg = jnp.float32(-jnp.inf)
    gv2 = jnp.where(sel1, neg, gv)
    mx2 = jnp.max(gv2, axis=1, keepdims=True)
    eq2 = jnp.logical_and(jnp.logical_not(sel1), gv2 == mx2)
    i2 = jnp.min(jnp.where(eq2, lane, E), axis=1, keepdims=True)
    sel2 = lane == i2
    w = jnp.where(jnp.logical_or(sel1, sel2), s, 0.0)
    sg = jax.nn.sigmoid(
        jax.lax.dot_general(x, sgw_ref[...], (((1,), (1,)), ((), ())),
                            preferred_element_type=jnp.float32))
    gv_ref[...] = gv
    waug = jnp.concatenate(
        [w, sg, jnp.zeros((x.shape[0], 16 - E - 1), jnp.float32)], axis=1)
    waug_ref[...] = waug


def _router(x, gate_w, shared_gate_w):
    T, D = x.shape
    E = gate_w.shape[0]
    Tb = 256
    return pl.pallas_call(
        _router_body,
        grid=(T // Tb,),
        in_specs=[
            pl.BlockSpec((Tb, D), lambda i: (i, 0)),
            pl.BlockSpec((E, D), lambda i: (0, 0)),
            pl.BlockSpec((1, D), lambda i: (0, 0)),
        ],
        out_specs=[
            pl.BlockSpec((Tb, E), lambda i: (i, 0)),
            pl.BlockSpec((Tb, 16), lambda i: (i, 0)),
        ],
        out_shape=[
            jax.ShapeDtypeStruct((T, E), jnp.float32),
            jax.ShapeDtypeStruct((T, 16), jnp.float32),
        ],
    )(x, gate_w, shared_gate_w)


def _experts_body(x_ref, wg_ref, wu_ref, wd_ref, waug_ref, out_ref):
    e = pl.program_id(0)
    j = pl.program_id(1)
    x = x_ref[...]
    g = jax.lax.dot_general(x, wg_ref[0], (((1,), (1,)), ((), ())),
                            preferred_element_type=jnp.float32)
    u = jax.lax.dot_general(x, wu_ref[0], (((1,), (1,)), ((), ())),
                            preferred_element_type=jnp.float32)
    h = (g * jax.nn.sigmoid(g)) * u
    part = jax.lax.dot_general(h, wd_ref[0], (((1,), (1,)), ((), ())),
                               preferred_element_type=jnp.float32)
    waug = waug_ref[...]
    lane = jax.lax.broadcasted_iota(jnp.int32, waug.shape, 1)
    wcol = jnp.sum(jnp.where(lane == e, waug, 0.0), axis=1, keepdims=True)
    part = part * wcol

    @pl.when(jnp.logical_and(e == 0, j == 0))
    def _():
        out_ref[...] = part

    @pl.when(jnp.logical_not(jnp.logical_and(e == 0, j == 0)))
    def _():
        out_ref[...] += part


def _experts_dense(x, Wg_aug, Wu_aug, Wd_aug, waug):
    T, D = x.shape
    EA, I, _ = Wg_aug.shape
    It = 128
    grid = (EA, I // It)
    return pl.pallas_call(
        _experts_body,
        grid=grid,
        in_specs=[
            pl.BlockSpec((T, D), lambda e, j: (0, 0)),
            pl.BlockSpec((1, It, D), lambda e, j: (e, j, 0)),
            pl.BlockSpec((1, It, D), lambda e, j: (e, j, 0)),
            pl.BlockSpec((1, D, It), lambda e, j: (e, 0, j)),
            pl.BlockSpec((T, 16), lambda e, j: (0, 0)),
        ],
        out_specs=pl.BlockSpec((T, D), lambda e, j: (0, 0)),
        out_shape=jax.ShapeDtypeStruct((T, D), jnp.float32),
        compiler_params=pltpu.CompilerParams(
            dimension_semantics=("arbitrary", "arbitrary"),
        ),
    )(x, Wg_aug, Wu_aug, Wd_aug, waug)


def kernel(hidden_states, gate_w, Wg, Wu, Wd, sWg, sWu, sWd, shared_gate_w):
    B, S, D = hidden_states.shape
    x = hidden_states.reshape(-1, D)
    gate_vals, waug = _router(x, gate_w, shared_gate_w)
    Wg_aug = jnp.concatenate([Wg, sWg[None]], axis=0)
    Wu_aug = jnp.concatenate([Wu, sWu[None]], axis=0)
    Wd_aug = jnp.concatenate([Wd, sWd[None]], axis=0)
    final = _experts_dense(x, Wg_aug, Wu_aug, Wd_aug, waug)
    return (final.reshape(B, S, D), gate_vals)
